# Initial kernel scaffold; baseline (speedup 1.0000x reference)
import jax
import jax.numpy as jnp
from jax.experimental import pallas as pl


def kernel(x_nchw, u1d1_w, u1d1_g, u1d1_b, uconv2_w, uconv2_g, uconv2_b, mconv1_w, mconv1_g, mconv1_b, u3d2_w, u3d2_g, u3d2_b, mconv2_w, mconv2_g, mconv2_b, uconv4_w, uconv4_g, uconv4_b, globalconv1_w, globalconv1_g, globalconv1_b, fc1_w, fc1_b, fc2_w, fc2_b):
    raise NotImplementedError("write your pallas kernel here")


# speedup vs baseline: 1.2998x; 1.2998x over previous
import numpy as np
import jax
import jax.numpy as jnp
from jax import lax
from jax.experimental import pallas as pl
from jax.experimental.pallas import tpu as pltpu

_EPS = 1e-5
_F32 = jnp.float32


def _bn_relu(acc, g, b, inv_m, mask=None):
    a = acc if mask is None else acc * mask
    mean = jnp.sum(a, axis=0, keepdims=True) * inv_m
    c = acc - mean
    cm = c if mask is None else c * mask
    var = jnp.sum(cm * cm, axis=0, keepdims=True) * inv_m
    y = c * (g * lax.rsqrt(var + _EPS)) + b
    y = jnp.maximum(y, 0.0)
    return y if mask is None else y * mask


def _make_gemm_bn(inv_m):
    def body(a_ref, w_ref, g_ref, b_ref, o_ref):
        acc = jnp.dot(a_ref[...], w_ref[...], preferred_element_type=_F32)
        y = _bn_relu(acc, g_ref[...], b_ref[...], inv_m)
        o_ref[...] = y[:, : o_ref.shape[1]]

    return body


def _make_shift_conv_bn(offsets, cin, guard, length, inv_m, pool):
    def body(a_ref, w_ref, g_ref, b_ref, mask_ref, *rest):
        if pool:
            p_ref, o_ref = rest
        else:
            (o_ref,) = rest
        acc = jnp.zeros((length, w_ref.shape[1]), _F32)
        for k, off in enumerate(offsets):
            lhs = a_ref[guard + off : guard + off + length, :]
            rhs = w_ref[k * cin : (k + 1) * cin, :]
            acc += jnp.dot(lhs, rhs, preferred_element_type=_F32)
        y = _bn_relu(acc, g_ref[...], b_ref[...], inv_m, mask_ref[...])
        if pool:
            o_ref[...] = jnp.dot(p_ref[...], y, preferred_element_type=_F32)
        else:
            o_ref[...] = y

    return body


def _fc_body(a_ref, w1_ref, b1_ref, w2_ref, b2_ref, o_ref):
    h = jnp.dot(a_ref[...], w1_ref[...], preferred_element_type=_F32)
    h = h + b1_ref[...]
    o_ref[...] = jnp.dot(h, w2_ref[...], preferred_element_type=_F32) + b2_ref[...]


def _full_call(body, operands, out_shape):
    return pl.pallas_call(
        body,
        out_shape=jax.ShapeDtypeStruct(out_shape, _F32),
        in_specs=[pl.BlockSpec(op.shape, lambda: (0,) * op.ndim)
                  for op in operands],
        out_specs=pl.BlockSpec(out_shape, lambda: (0,) * len(out_shape)),
    )(*operands)


def _nsplit_call(body, operands, split_in, out_shape):
    in_specs = []
    for op, split in zip(operands, split_in):
        if split:
            blk = (op.shape[0], op.shape[1] // 2)
            in_specs.append(pl.BlockSpec(blk, lambda j: (0, j)))
        else:
            in_specs.append(pl.BlockSpec(op.shape, lambda j: (0, 0)))
    out_blk = (out_shape[0], out_shape[1] // 2)
    return pl.pallas_call(
        body,
        out_shape=jax.ShapeDtypeStruct(out_shape, _F32),
        grid=(2,),
        in_specs=in_specs,
        out_specs=pl.BlockSpec(out_blk, lambda j: (0, j)),
        compiler_params=pltpu.CompilerParams(
            dimension_semantics=("parallel",)),
    )(*operands)


def _im2col_s2(x, ho):
    taps = [x[:, dh:dh + 2 * ho - 1:2, dw:dw + 2 * ho - 1:2, :]
            for dh in range(3) for dw in range(3)]
    p = jnp.concatenate(taps, axis=-1)
    return p.reshape(-1, p.shape[-1])


def _flat_pad(x4, guard):
    n, hp, wp, c = x4.shape
    flat = x4.reshape(n * hp * wp, c)
    return jnp.pad(flat, ((guard, guard), (0, 0)))


def _valid_mask(n, hp, ho):
    q = np.arange(n * hp * hp) % (hp * hp)
    i, j = q // hp, q % hp
    ok = (i >= 1) & (i <= ho) & (j >= 1) & (j <= ho)
    return jnp.asarray(ok.astype(np.float32)[:, None])


def _pool_matrix(n, hp, h):
    p = np.zeros((n * 4, n * hp * hp), np.float32)
    half = [(0, -(-h // 2)), (h // 2, h)]
    for img in range(n):
        for i, (r0, r1) in enumerate(half):
            for j, (c0, c1) in enumerate(half):
                inv = 1.0 / ((r1 - r0) * (c1 - c0))
                for r in range(r0, r1):
                    for c in range(c0, c1):
                        p[img * 4 + i * 2 + j,
                          img * hp * hp + (r + 1) * hp + (c + 1)] = inv
    return jnp.asarray(p)


def kernel(x_nchw,
           u1d1_w, u1d1_g, u1d1_b,
           uconv2_w, uconv2_g, uconv2_b,
           mconv1_w, mconv1_g, mconv1_b,
           u3d2_w, u3d2_g, u3d2_b,
           mconv2_w, mconv2_g, mconv2_b,
           uconv4_w, uconv4_g, uconv4_b,
           globalconv1_w, globalconv1_g, globalconv1_b,
           fc1_w, fc1_b, fc2_w, fc2_b):
    n = x_nchw.shape[0]
    x = jnp.transpose(x_nchw, (0, 2, 3, 1))
    xp = jnp.pad(x, ((0, 0), (1, 1), (1, 1), (0, 0)))

    taps1 = [xp[:, dh:dh + 34, dw:dw + 34, 0] for dh in range(3)
             for dw in range(3)]
    a1 = jnp.stack(taps1, axis=-1).reshape(n * 34 * 34, 9)
    a1 = jnp.pad(a1, ((0, 0), (0, 7)))
    ud = _full_call(_make_gemm_bn(1.0 / (n * 34 * 34)),
                    (a1, u1d1_w[:16, :], u1d1_g, u1d1_b),
                    (n * 34 * 34, 48))
    ud4 = ud.reshape(n, 34, 34, 48)

    udp = jnp.pad(ud4[..., :16], ((0, 0), (1, 1), (1, 1), (0, 0)))
    p_u = _im2col_s2(udp, 17)
    p_m = _im2col_s2(xp, 17)
    a2 = jnp.zeros((n * 289, 160), _F32)
    a2 = a2.at[:, :144].set(p_u).at[:, 144:153].set(p_m)
    w2 = jnp.zeros((160, 128), _F32)
    w2 = w2.at[:144, :32].set(uconv2_w[:144, :32])
    w2 = w2.at[144:153, 32:64].set(mconv1_w[:9, :32])
    g2 = jnp.zeros((1, 128), _F32).at[0, :32].set(uconv2_g[0, :32]) \
        .at[0, 32:64].set(mconv1_g[0, :32])
    b2 = jnp.zeros((1, 128), _F32).at[0, :32].set(uconv2_b[0, :32]) \
        .at[0, 32:64].set(mconv1_b[0, :32])
    o2 = _full_call(_make_gemm_bn(1.0 / (n * 289)), (a2, w2, g2, b2),
                    (n * 289, 64))
    uout = o2[:, :32].reshape(n, 17, 17, 32)
    mout = o2[:, 32:64].reshape(n, 17, 17, 32)
    dout = ud4[..., 16:48].reshape(n, 17, 2, 17, 2, 32).max(axis=(2, 4))
    out = jnp.concatenate([uout, mout, dout], axis=-1)

    outp = jnp.pad(out, ((0, 0), (1, 1), (1, 1), (0, 0)))
    l3 = n * 19 * 19
    offs19 = [(dh - 1) * 19 + (dw - 1) for dh in range(3) for dw in range(3)]
    a3 = _flat_pad(outp, 20)
    m3 = _valid_mask(n, 19, 17)
    out2p = _nsplit_call(
        _make_shift_conv_bn(offs19, 96, 20, l3, 1.0 / (n * 289), False),
        (a3, u3d2_w, u3d2_g, u3d2_b, m3),
        (False, True, True, True, False),
        (l3, 256))
    out2_4 = out2p.reshape(n, 19, 19, 256)

    a4 = _im2col_s2(outp, 9)
    mout2 = _full_call(_make_gemm_bn(1.0 / (n * 81)),
                       (a4, mconv2_w[:864, :], mconv2_g, mconv2_b),
                       (n * 81, 128))

    a5 = _im2col_s2(out2_4, 9)
    uout4 = _nsplit_call(_make_gemm_bn(1.0 / (n * 81)),
                         (a5, uconv4_w, uconv4_g, uconv4_b),
                         (False, True, True, True),
                         (n * 81, 512))
    dout2 = out2_4[:, :18, :18, :].reshape(n, 9, 2, 9, 2, 256).max(axis=(2, 4))
    out3 = jnp.concatenate([uout4.reshape(n, 9, 9, 512),
                            mout2.reshape(n, 9, 9, 128),
                            dout2], axis=-1)

    out3p = jnp.pad(out3, ((0, 0), (1, 1), (1, 1), (0, 0)))
    l5 = n * 11 * 11
    offs11 = [(dh - 1) * 11 + (dw - 1) for dh in range(3) for dw in range(3)]
    a6 = _flat_pad(out3p, 12)
    m5 = _valid_mask(n, 11, 9)
    pmat = _pool_matrix(n, 11, 9)
    pooled = _nsplit_call(
        _make_shift_conv_bn(offs11, 896, 12, l5, 1.0 / (n * 81), True),
        (a6, globalconv1_w, globalconv1_g, globalconv1_b, m5, pmat),
        (False, True, True, True, False, False),
        (n * 4, 1024))

    flat = pooled.reshape(n, 4, 1024).transpose(0, 2, 1).reshape(n, 4096)
    y = _full_call(_fc_body, (flat, fc1_w, fc1_b, fc2_w, fc2_b), (n, 128))
    return y[:, :10]


# speedup vs baseline: 4.6129x; 3.5490x over previous
import numpy as np
import jax
import jax.numpy as jnp
from jax import lax
from jax.experimental import pallas as pl
from jax.experimental.pallas import tpu as pltpu

_EPS = 1e-5
_F32 = jnp.float32

_L1, _G1 = 16 * 36 * 36, 37
_L3, _G3 = 16 * 19 * 19, 20
_L5, _G5 = 16 * 11 * 11, 12
_OFF36 = [(dh - 1) * 36 + (dw - 1) for dh in range(3) for dw in range(3)]
_OFF19 = [(dh - 1) * 19 + (dw - 1) for dh in range(3) for dw in range(3)]
_OFF11 = [(dh - 1) * 11 + (dw - 1) for dh in range(3) for dw in range(3)]


def _bn_relu(acc, g, b, inv_m, mask):
    am = acc * mask
    mean = jnp.sum(am, axis=0, keepdims=True) * inv_m
    c = acc - mean
    cm = c * mask
    var = jnp.sum(cm * cm, axis=0, keepdims=True) * inv_m
    y = c * (g * lax.rsqrt(var + _EPS)) + b
    return jnp.maximum(y, 0.0) * mask


def _sel16(s_ref, acc, rows_in):
    s = s_ref[...]
    return jnp.concatenate(
        [jnp.dot(s, acc[i * rows_in:(i + 1) * rows_in, :],
                 preferred_element_type=_F32) for i in range(16)], axis=0)


def _gemm_bn_body(a_ref, w_ref, g_ref, b_ref, o_ref):
    acc = jnp.dot(a_ref[...], w_ref[...], preferred_element_type=_F32)
    inv_m = 1.0 / a_ref.shape[0]
    mean = jnp.sum(acc, axis=0, keepdims=True) * inv_m
    c = acc - mean
    var = jnp.sum(c * c, axis=0, keepdims=True) * inv_m
    y = c * (g_ref[...] * lax.rsqrt(var + _EPS)) + b_ref[...]
    o_ref[...] = jnp.maximum(y, 0.0)[:, :o_ref.shape[1]]


def _c1b_body(out_ref, w3_ref, w4_ref,
              g3_ref, b3_ref, g4_ref, b4_ref,
              s9c_ref, s9tl_ref, m19_ref, m11_ref, o_ref, out2_ref):
    m19, m11 = m19_ref[...], m11_ref[...]

    acc3 = jnp.zeros((_L3, 256), _F32)
    for k, off in enumerate(_OFF19):
        acc3 += jnp.dot(out_ref[_G3 + off:_G3 + off + _L3, :],
                        w3_ref[k * 96:(k + 1) * 96, :],
                        preferred_element_type=_F32)
    out2_ref[...] = jnp.zeros((_L3 + 2 * _G3, 256), _F32)
    out2_ref[_G3:_G3 + _L3, :] = _bn_relu(acc3, g3_ref[...], b3_ref[...],
                                          1.0 / 4624.0, m19)

    mpa = jnp.maximum(
        jnp.maximum(out2_ref[_G3:_G3 + _L3, :],
                    out2_ref[_G3 + 1:_G3 + 1 + _L3, :]),
        jnp.maximum(out2_ref[_G3 + 19:_G3 + 19 + _L3, :],
                    out2_ref[_G3 + 20:_G3 + 20 + _L3, :]))
    d2 = _sel16(s9tl_ref, mpa, 361)

    acc4 = jnp.zeros((_L3, 512), _F32)
    for k, off in enumerate(_OFF19):
        acc4 += jnp.dot(out2_ref[_G3 + off:_G3 + off + _L3, :],
                        w4_ref[k * 256:(k + 1) * 256, :],
                        preferred_element_type=_F32)
    u4_bn = _bn_relu(_sel16(s9c_ref, acc4, 361), g4_ref[...], b4_ref[...],
                     1.0 / 1296.0, m11)

    o_ref[...] = jnp.zeros((_L5 + 2 * _G5, 768), _F32)
    o_ref[_G5:_G5 + _L5, 0:512] = u4_bn
    o_ref[_G5:_G5 + _L5, 512:768] = d2


def _c1c_body(out_ref, wm2_ref, gm2_ref, bm2_ref, s9c_ref, m11_ref, o_ref):
    accm2 = jnp.zeros((_L3, 128), _F32)
    for k, off in enumerate(_OFF19):
        accm2 += jnp.dot(out_ref[_G3 + off:_G3 + off + _L3, :],
                         wm2_ref[k * 96:(k + 1) * 96, :],
                         preferred_element_type=_F32)
    m2_bn = _bn_relu(_sel16(s9c_ref, accm2, 361), gm2_ref[...], bm2_ref[...],
                     1.0 / 1296.0, m11_ref[...])
    o_ref[...] = jnp.zeros((_L5 + 2 * _G5, 128), _F32)
    o_ref[_G5:_G5 + _L5, :] = m2_bn


def _c2_body(oa_ref, ob_ref, w_ref, g_ref, b_ref, m_ref, p_ref, o_ref):
    acc = jnp.zeros((_L5, w_ref.shape[1]), _F32)
    for k, off in enumerate(_OFF11):
        sl = slice(_G5 + off, _G5 + off + _L5)
        base = k * 896
        acc += jnp.dot(oa_ref[sl, 0:512], w_ref[base:base + 512, :],
                       preferred_element_type=_F32)
        acc += jnp.dot(ob_ref[sl, :], w_ref[base + 512:base + 640, :],
                       preferred_element_type=_F32)
        acc += jnp.dot(oa_ref[sl, 512:768], w_ref[base + 640:base + 896, :],
                       preferred_element_type=_F32)
    y = _bn_relu(acc, g_ref[...], b_ref[...], 1.0 / 1296.0, m_ref[...])
    o_ref[...] = jnp.dot(p_ref[...], y, preferred_element_type=_F32)


def _c3_body(a_ref, w1_ref, b1_ref, w2_ref, b2_ref, o_ref):
    h = jnp.dot(a_ref[...], w1_ref[...], preferred_element_type=_F32)
    h = h + b1_ref[...]
    y = jnp.dot(h, w2_ref[...], preferred_element_type=_F32) + b2_ref[...]
    o_ref[...] = y[:, :10]


def _full_specs(operands):
    return [pl.BlockSpec(op.shape, lambda: (0,) * op.ndim) for op in operands]


def _sel_center(hp_out, hp_in):
    ho = (hp_in - 2 - 1) // 2 + 1
    s = np.zeros((hp_out * hp_out, hp_in * hp_in), np.float32)
    for i in range(ho):
        for j in range(ho):
            s[(i + 1) * hp_out + (j + 1), (2 * i + 1) * hp_in + (2 * j + 1)] = 1.0
    return jnp.asarray(s)


def _sel_topleft(hp_out, hp_in, ho):
    s = np.zeros((hp_out * hp_out, hp_in * hp_in), np.float32)
    for i in range(ho):
        for j in range(ho):
            s[(i + 1) * hp_out + (j + 1), (2 * i) * hp_in + (2 * j)] = 1.0
    return jnp.asarray(s)


def _interior_mask(n, hp, ho):
    q = np.arange(n * hp * hp) % (hp * hp)
    i, j = q // hp, q % hp
    ok = (i >= 1) & (i <= ho) & (j >= 1) & (j <= ho)
    return jnp.asarray(ok.astype(np.float32)[:, None])


def _pool_matrix(n, hp, h):
    p = np.zeros((n * 4, n * hp * hp), np.float32)
    half = [(0, -(-h // 2)), (h // 2, h)]
    for img in range(n):
        for i, (r0, r1) in enumerate(half):
            for j, (c0, c1) in enumerate(half):
                inv = 1.0 / ((r1 - r0) * (c1 - c0))
                for r in range(r0, r1):
                    for c in range(c0, c1):
                        p[img * 4 + i * 2 + j,
                          img * hp * hp + (r + 1) * hp + (c + 1)] = inv
    return jnp.asarray(p)


def kernel(x_nchw,
           u1d1_w, u1d1_g, u1d1_b,
           uconv2_w, uconv2_g, uconv2_b,
           mconv1_w, mconv1_g, mconv1_b,
           u3d2_w, u3d2_g, u3d2_b,
           mconv2_w, mconv2_g, mconv2_b,
           uconv4_w, uconv4_g, uconv4_b,
           globalconv1_w, globalconv1_g, globalconv1_b,
           fc1_w, fc1_b, fc2_w, fc2_b):
    n = x_nchw.shape[0]

    xi = jnp.pad(x_nchw.reshape(n, 34, 34, 1), ((0, 0), (1, 1), (1, 1), (0, 0)))
    taps1 = [xi[:, dh:dh + 34, dw:dw + 34, :] for dh in range(3)
             for dw in range(3)]
    a1 = jnp.concatenate(taps1, axis=-1).reshape(n * 34 * 34, 9)
    a1 = jnp.pad(a1, ((0, 0), (0, 7)))
    a1_ops = (a1, u1d1_w[:16, :48], u1d1_g[:, :48], u1d1_b[:, :48])
    ud = pl.pallas_call(
        _gemm_bn_body,
        out_shape=jax.ShapeDtypeStruct((n * 34 * 34, 48), _F32),
        in_specs=_full_specs(a1_ops),
        out_specs=pl.BlockSpec((n * 34 * 34, 48), lambda: (0, 0)),
    )(*a1_ops)
    ud4 = ud.reshape(n, 34, 34, 48)

    udp = jnp.pad(ud4[..., :16], ((0, 0), (1, 1), (1, 1), (0, 0)))
    p_u = [udp[:, dh:dh + 33:2, dw:dw + 33:2, :] for dh in range(3)
           for dw in range(3)]
    p_m = [xi[:, dh:dh + 33:2, dw:dw + 33:2, :] for dh in range(3)
           for dw in range(3)]
    a2 = jnp.concatenate(p_u + p_m, axis=-1).reshape(n * 289, 153)
    a2 = jnp.pad(a2, ((0, 0), (0, 7)))
    w2 = jnp.zeros((160, 64), _F32)
    w2 = w2.at[:144, :32].set(uconv2_w[:144, :32])
    w2 = w2.at[144:153, 32:64].set(mconv1_w[:9, :32])
    g2 = jnp.concatenate([uconv2_g[:, :32], mconv1_g[:, :32]], axis=1)
    b2 = jnp.concatenate([uconv2_b[:, :32], mconv1_b[:, :32]], axis=1)
    a2_ops = (a2, w2, g2, b2)
    o2 = pl.pallas_call(
        _gemm_bn_body,
        out_shape=jax.ShapeDtypeStruct((n * 289, 64), _F32),
        in_specs=_full_specs(a2_ops),
        out_specs=pl.BlockSpec((n * 289, 64), lambda: (0, 0)),
    )(*a2_ops)
    uout = o2[:, :32].reshape(n, 17, 17, 32)
    mout = o2[:, 32:64].reshape(n, 17, 17, 32)
    dout = ud4[..., 16:48].reshape(n, 17, 2, 17, 2, 32).max(axis=(2, 4))
    out = jnp.concatenate([uout, mout, dout], axis=-1)
    outp = jnp.pad(out, ((0, 0), (1, 1), (1, 1), (0, 0)))
    out_map = jnp.pad(outp.reshape(_L3, 96), ((_G3, _G3), (0, 0)))

    m19 = _interior_mask(n, 19, 17)
    m11 = _interior_mask(n, 11, 9)

    s9c = _sel_center(11, 19)
    c1b_ops = (out_map, u3d2_w, uconv4_w,
               u3d2_g, u3d2_b, uconv4_g, uconv4_b,
               s9c, _sel_topleft(11, 19, 9), m19, m11)
    out3a = pl.pallas_call(
        _c1b_body,
        out_shape=jax.ShapeDtypeStruct((_L5 + 2 * _G5, 768), _F32),
        in_specs=_full_specs(c1b_ops),
        out_specs=pl.BlockSpec((_L5 + 2 * _G5, 768), lambda: (0, 0)),
        scratch_shapes=[pltpu.VMEM((_L3 + 2 * _G3, 256), _F32)],
    )(*c1b_ops)

    c1c_ops = (out_map, mconv2_w, mconv2_g, mconv2_b, s9c, m11)
    out3b = pl.pallas_call(
        _c1c_body,
        out_shape=jax.ShapeDtypeStruct((_L5 + 2 * _G5, 128), _F32),
        in_specs=_full_specs(c1c_ops),
        out_specs=pl.BlockSpec((_L5 + 2 * _G5, 128), lambda: (0, 0)),
    )(*c1c_ops)

    c2_ops = (out3a, out3b, globalconv1_w, globalconv1_g, globalconv1_b,
              m11, _pool_matrix(n, 11, 9))
    pooled = pl.pallas_call(
        _c2_body,
        out_shape=jax.ShapeDtypeStruct((n * 4, 1024), _F32),
        grid=(2,),
        in_specs=[
            pl.BlockSpec(out3a.shape, lambda j: (0, 0)),
            pl.BlockSpec(out3b.shape, lambda j: (0, 0)),
            pl.BlockSpec((8192, 512), lambda j: (0, j)),
            pl.BlockSpec((1, 512), lambda j: (0, j)),
            pl.BlockSpec((1, 512), lambda j: (0, j)),
            pl.BlockSpec((_L5, 1), lambda j: (0, 0)),
            pl.BlockSpec((n * 4, _L5), lambda j: (0, 0)),
        ],
        out_specs=pl.BlockSpec((n * 4, 512), lambda j: (0, j)),
        compiler_params=pltpu.CompilerParams(
            dimension_semantics=("parallel",)),
    )(*c2_ops)

    flat = pooled.reshape(n, 4, 1024).transpose(0, 2, 1).reshape(n, 4096)
    c3_ops = (flat, fc1_w, fc1_b, fc2_w, fc2_b)
    return pl.pallas_call(
        _c3_body,
        out_shape=jax.ShapeDtypeStruct((n, 10), _F32),
        in_specs=_full_specs(c3_ops),
        out_specs=pl.BlockSpec((n, 10), lambda: (0, 0)),
    )(*c3_ops)


# speedup vs baseline: 10.6942x; 2.3183x over previous
import numpy as np
import jax
import jax.numpy as jnp
from jax import lax
from jax.experimental import pallas as pl
from jax.experimental.pallas import tpu as pltpu

_EPS = 1e-5
_F32 = jnp.float32

_L1, _G1 = 16 * 36 * 36, 37
_L3, _G3 = 16 * 19 * 19, 20
_L5, _G5 = 16 * 11 * 11, 12
_OFF36 = [(dh - 1) * 36 + (dw - 1) for dh in range(3) for dw in range(3)]
_OFF19 = [(dh - 1) * 19 + (dw - 1) for dh in range(3) for dw in range(3)]
_OFF11 = [(dh - 1) * 11 + (dw - 1) for dh in range(3) for dw in range(3)]


def _bn_relu(acc, g, b, inv_m, mask):
    am = acc * mask
    mean = jnp.sum(am, axis=0, keepdims=True) * inv_m
    c = acc - mean
    cm = c * mask
    var = jnp.sum(cm * cm, axis=0, keepdims=True) * inv_m
    y = c * (g * lax.rsqrt(var + _EPS)) + b
    return jnp.maximum(y, 0.0) * mask


def _sel16(s_ref, acc, rows_in):
    s = s_ref[...]
    return jnp.concatenate(
        [jnp.dot(s, acc[i * rows_in:(i + 1) * rows_in, :],
                 preferred_element_type=_F32) for i in range(16)], axis=0)


def _c1a_body(x_ref, w1_ref, w2_ref, wm1_ref,
              g1_ref, b1_ref, g2_ref, b2_ref, gm1_ref, bm1_ref,
              s17_ref, m1296_ref, m361_ref, o_ref, ud_ref):
    gr = 4 * 1296
    mg = jnp.concatenate([m1296_ref[...]] * 4, axis=0)

    s1 = jnp.zeros((1, 48), _F32)
    s2 = jnp.zeros((1, 48), _F32)
    for g in range(4):
        base = _G1 + g * gr
        acc = jnp.zeros((gr, 48), _F32)
        for k, off in enumerate(_OFF36):
            acc += jnp.dot(x_ref[base + off:base + off + gr, :],
                           w1_ref[k:k + 1, 0:48], preferred_element_type=_F32)
        am = acc * mg
        s1 += jnp.sum(am, axis=0, keepdims=True)
        s2 += jnp.sum(am * acc, axis=0, keepdims=True)
        ud_ref[base:base + gr, :] = acc
    mean = s1 * (1.0 / 18496.0)
    var = s2 * (1.0 / 18496.0) - mean * mean
    scale = g1_ref[:, 0:48] * lax.rsqrt(var + _EPS)
    beta = b1_ref[:, 0:48]

    for g in range(4):
        base = _G1 + g * gr
        a = ud_ref[base:base + gr, :]
        ud_ref[base:base + gr, :] = jnp.maximum((a - mean) * scale + beta,
                                                0.0) * mg
    ud_ref[0:_G1, :] = jnp.zeros((_G1, 48), _F32)
    ud_ref[_G1 + _L1:_G1 + _L1 + _G1, :] = jnp.zeros((_G1, 48), _F32)

    s17 = s17_ref[...]
    pu, pm, pd = [], [], []
    for g in range(4):
        base = _G1 + g * gr
        accu = jnp.zeros((gr, 32), _F32)
        accm = jnp.zeros((gr, 32), _F32)
        for k, off in enumerate(_OFF36):
            accu += jnp.dot(ud_ref[base + off:base + off + gr, 0:16],
                            w2_ref[k * 16:(k + 1) * 16, 0:32],
                            preferred_element_type=_F32)
            accm += jnp.dot(x_ref[base + off:base + off + gr, :],
                            wm1_ref[k:k + 1, 0:32],
                            preferred_element_type=_F32)
        mp = jnp.maximum(
            jnp.maximum(ud_ref[base:base + gr, 16:48],
                        ud_ref[base + 1:base + 1 + gr, 16:48]),
            jnp.maximum(ud_ref[base + 36:base + 36 + gr, 16:48],
                        ud_ref[base + 37:base + 37 + gr, 16:48]))
        for i in range(4):
            sl = slice(i * 1296, (i + 1) * 1296)
            pu.append(jnp.dot(s17, accu[sl, :], preferred_element_type=_F32))
            pm.append(jnp.dot(s17, accm[sl, :], preferred_element_type=_F32))
            pd.append(jnp.dot(s17, mp[sl, :], preferred_element_type=_F32))
    m19 = jnp.concatenate([m361_ref[...]] * 16, axis=0)
    u_bn = _bn_relu(jnp.concatenate(pu, axis=0), g2_ref[:, 0:32],
                    b2_ref[:, 0:32], 1.0 / 4624.0, m19)
    m_bn = _bn_relu(jnp.concatenate(pm, axis=0), gm1_ref[:, 0:32],
                    bm1_ref[:, 0:32], 1.0 / 4624.0, m19)
    o_ref[...] = jnp.zeros((_L3 + 2 * _G3, 96), _F32)
    o_ref[_G3:_G3 + _L3, 0:32] = u_bn
    o_ref[_G3:_G3 + _L3, 32:64] = m_bn
    o_ref[_G3:_G3 + _L3, 64:96] = jnp.concatenate(pd, axis=0)


def _c1b_body(out_ref, w3_ref, w4_ref,
              g3_ref, b3_ref, g4_ref, b4_ref,
              s9c_ref, s9tl_ref, m19_ref, m11_ref, o_ref, out2_ref):
    m19, m11 = m19_ref[...], m11_ref[...]

    acc3 = jnp.zeros((_L3, 256), _F32)
    for k, off in enumerate(_OFF19):
        acc3 += jnp.dot(out_ref[_G3 + off:_G3 + off + _L3, :],
                        w3_ref[k * 96:(k + 1) * 96, :],
                        preferred_element_type=_F32)
    out2_ref[...] = jnp.zeros((_L3 + 2 * _G3, 256), _F32)
    out2_ref[_G3:_G3 + _L3, :] = _bn_relu(acc3, g3_ref[...], b3_ref[...],
                                          1.0 / 4624.0, m19)

    mpa = jnp.maximum(
        jnp.maximum(out2_ref[_G3:_G3 + _L3, :],
                    out2_ref[_G3 + 1:_G3 + 1 + _L3, :]),
        jnp.maximum(out2_ref[_G3 + 19:_G3 + 19 + _L3, :],
                    out2_ref[_G3 + 20:_G3 + 20 + _L3, :]))
    d2 = _sel16(s9tl_ref, mpa, 361)

    acc4 = jnp.zeros((_L3, 512), _F32)
    for k, off in enumerate(_OFF19):
        acc4 += jnp.dot(out2_ref[_G3 + off:_G3 + off + _L3, :],
                        w4_ref[k * 256:(k + 1) * 256, :],
                        preferred_element_type=_F32)
    u4_bn = _bn_relu(_sel16(s9c_ref, acc4, 361), g4_ref[...], b4_ref[...],
                     1.0 / 1296.0, m11)

    o_ref[...] = jnp.zeros((_L5 + 2 * _G5, 768), _F32)
    o_ref[_G5:_G5 + _L5, 0:512] = u4_bn
    o_ref[_G5:_G5 + _L5, 512:768] = d2


def _c1c_body(out_ref, wm2_ref, gm2_ref, bm2_ref, s9c_ref, m11_ref, o_ref):
    accm2 = jnp.zeros((_L3, 128), _F32)
    for k, off in enumerate(_OFF19):
        accm2 += jnp.dot(out_ref[_G3 + off:_G3 + off + _L3, :],
                         wm2_ref[k * 96:(k + 1) * 96, :],
                         preferred_element_type=_F32)
    m2_bn = _bn_relu(_sel16(s9c_ref, accm2, 361), gm2_ref[...], bm2_ref[...],
                     1.0 / 1296.0, m11_ref[...])
    o_ref[...] = jnp.zeros((_L5 + 2 * _G5, 128), _F32)
    o_ref[_G5:_G5 + _L5, :] = m2_bn


def _c2_body(oa_ref, ob_ref, w_ref, g_ref, b_ref, m_ref, p_ref, o_ref):
    acc = jnp.zeros((_L5, w_ref.shape[1]), _F32)
    for k, off in enumerate(_OFF11):
        sl = slice(_G5 + off, _G5 + off + _L5)
        base = k * 896
        acc += jnp.dot(oa_ref[sl, 0:512], w_ref[base:base + 512, :],
                       preferred_element_type=_F32)
        acc += jnp.dot(ob_ref[sl, :], w_ref[base + 512:base + 640, :],
                       preferred_element_type=_F32)
        acc += jnp.dot(oa_ref[sl, 512:768], w_ref[base + 640:base + 896, :],
                       preferred_element_type=_F32)
    y = _bn_relu(acc, g_ref[...], b_ref[...], 1.0 / 1296.0, m_ref[...])
    o_ref[...] = jnp.dot(p_ref[...], y, preferred_element_type=_F32)


def _c3_body(a_ref, w1_ref, b1_ref, w2_ref, b2_ref, o_ref):
    h = jnp.dot(a_ref[...], w1_ref[...], preferred_element_type=_F32)
    h = h + b1_ref[...]
    y = jnp.dot(h, w2_ref[...], preferred_element_type=_F32) + b2_ref[...]
    o_ref[...] = y[:, :10]


def _full_specs(operands):
    return [pl.BlockSpec(op.shape, lambda: (0,) * op.ndim) for op in operands]


def _sel_center(hp_out, hp_in):
    ho = (hp_in - 2 - 1) // 2 + 1
    s = np.zeros((hp_out * hp_out, hp_in * hp_in), np.float32)
    for i in range(ho):
        for j in range(ho):
            s[(i + 1) * hp_out + (j + 1), (2 * i + 1) * hp_in + (2 * j + 1)] = 1.0
    return jnp.asarray(s)


def _sel_topleft(hp_out, hp_in, ho):
    s = np.zeros((hp_out * hp_out, hp_in * hp_in), np.float32)
    for i in range(ho):
        for j in range(ho):
            s[(i + 1) * hp_out + (j + 1), (2 * i) * hp_in + (2 * j)] = 1.0
    return jnp.asarray(s)


def _interior_mask1(hp, ho):
    q = np.arange(hp * hp)
    i, j = q // hp, q % hp
    ok = (i >= 1) & (i <= ho) & (j >= 1) & (j <= ho)
    return jnp.asarray(ok.astype(np.float32)[:, None])


def _interior_mask(n, hp, ho):
    q = np.arange(n * hp * hp) % (hp * hp)
    i, j = q // hp, q % hp
    ok = (i >= 1) & (i <= ho) & (j >= 1) & (j <= ho)
    return jnp.asarray(ok.astype(np.float32)[:, None])


def _pool_matrix(n, hp, h):
    p = np.zeros((n * 4, n * hp * hp), np.float32)
    half = [(0, -(-h // 2)), (h // 2, h)]
    for img in range(n):
        for i, (r0, r1) in enumerate(half):
            for j, (c0, c1) in enumerate(half):
                inv = 1.0 / ((r1 - r0) * (c1 - c0))
                for r in range(r0, r1):
                    for c in range(c0, c1):
                        p[img * 4 + i * 2 + j,
                          img * hp * hp + (r + 1) * hp + (c + 1)] = inv
    return jnp.asarray(p)


def kernel(x_nchw,
           u1d1_w, u1d1_g, u1d1_b,
           uconv2_w, uconv2_g, uconv2_b,
           mconv1_w, mconv1_g, mconv1_b,
           u3d2_w, u3d2_g, u3d2_b,
           mconv2_w, mconv2_g, mconv2_b,
           uconv4_w, uconv4_g, uconv4_b,
           globalconv1_w, globalconv1_g, globalconv1_b,
           fc1_w, fc1_b, fc2_w, fc2_b):
    n = x_nchw.shape[0]

    xi = jnp.pad(x_nchw.reshape(n, 34, 34), ((0, 0), (1, 1), (1, 1)))
    x_pre = jnp.pad(xi.reshape(n * 36 * 36, 1), ((_G1, _G1), (0, 0)))

    c1a_ops = (x_pre, u1d1_w, uconv2_w, mconv1_w,
               u1d1_g, u1d1_b, uconv2_g, uconv2_b, mconv1_g, mconv1_b,
               _sel_center(19, 36), _interior_mask1(36, 34),
               _interior_mask1(19, 17))
    out_map = pl.pallas_call(
        _c1a_body,
        out_shape=jax.ShapeDtypeStruct((_L3 + 2 * _G3, 96), _F32),
        in_specs=_full_specs(c1a_ops),
        out_specs=pl.BlockSpec((_L3 + 2 * _G3, 96), lambda: (0, 0)),
        scratch_shapes=[pltpu.VMEM((_L1 + 2 * _G1, 48), _F32)],
    )(*c1a_ops)

    m19 = _interior_mask(n, 19, 17)
    m11 = _interior_mask(n, 11, 9)

    s9c = _sel_center(11, 19)
    c1b_ops = (out_map, u3d2_w, uconv4_w,
               u3d2_g, u3d2_b, uconv4_g, uconv4_b,
               s9c, _sel_topleft(11, 19, 9), m19, m11)
    out3a = pl.pallas_call(
        _c1b_body,
        out_shape=jax.ShapeDtypeStruct((_L5 + 2 * _G5, 768), _F32),
        in_specs=_full_specs(c1b_ops),
        out_specs=pl.BlockSpec((_L5 + 2 * _G5, 768), lambda: (0, 0)),
        scratch_shapes=[pltpu.VMEM((_L3 + 2 * _G3, 256), _F32)],
    )(*c1b_ops)

    c1c_ops = (out_map, mconv2_w, mconv2_g, mconv2_b, s9c, m11)
    out3b = pl.pallas_call(
        _c1c_body,
        out_shape=jax.ShapeDtypeStruct((_L5 + 2 * _G5, 128), _F32),
        in_specs=_full_specs(c1c_ops),
        out_specs=pl.BlockSpec((_L5 + 2 * _G5, 128), lambda: (0, 0)),
    )(*c1c_ops)

    c2_ops = (out3a, out3b, globalconv1_w, globalconv1_g, globalconv1_b,
              m11, _pool_matrix(n, 11, 9))
    pooled = pl.pallas_call(
        _c2_body,
        out_shape=jax.ShapeDtypeStruct((n * 4, 1024), _F32),
        grid=(2,),
        in_specs=[
            pl.BlockSpec(out3a.shape, lambda j: (0, 0)),
            pl.BlockSpec(out3b.shape, lambda j: (0, 0)),
            pl.BlockSpec((8192, 512), lambda j: (0, j)),
            pl.BlockSpec((1, 512), lambda j: (0, j)),
            pl.BlockSpec((1, 512), lambda j: (0, j)),
            pl.BlockSpec((_L5, 1), lambda j: (0, 0)),
            pl.BlockSpec((n * 4, _L5), lambda j: (0, 0)),
        ],
        out_specs=pl.BlockSpec((n * 4, 512), lambda j: (0, j)),
        compiler_params=pltpu.CompilerParams(
            dimension_semantics=("parallel",)),
    )(*c2_ops)

    flat = pooled.reshape(n, 4, 1024).transpose(0, 2, 1).reshape(n, 4096)
    c3_ops = (flat, fc1_w, fc1_b, fc2_w, fc2_b)
    return pl.pallas_call(
        _c3_body,
        out_shape=jax.ShapeDtypeStruct((n, 10), _F32),
        in_specs=_full_specs(c3_ops),
        out_specs=pl.BlockSpec((n, 10), lambda: (0, 0)),
    )(*c3_ops)
